# Initial kernel scaffold; baseline (speedup 1.0000x reference)
import jax
import jax.numpy as jnp
from jax.experimental import pallas as pl


def kernel(points_3D, embeddings):
    raise NotImplementedError("write your pallas kernel here")


# speedup vs baseline: 1.4525x; 1.4525x over previous
import jax
import jax.numpy as jnp
import numpy as np
from jax import lax
from jax.experimental import pallas as pl
from jax.experimental.pallas import tpu as pltpu
from jax.experimental.pallas import tpu_sc as plsc

_NUM_LEVELS = 16
_LEVEL_DIM = 4
_LOG2_T = 19
_T = 2 ** _LOG2_T
_BASE_RES = 16
_MAX_RES = 2048
_GROWTH = np.exp((np.log(_MAX_RES) - np.log(_BASE_RES)) / (_NUM_LEVELS - 1))
_RES = [int(np.floor(_BASE_RES * _GROWTH ** l)) + 1 for l in range(_NUM_LEVELS)]
_SIZES = [min(_T, r ** 3) for r in _RES]
_OFFS = np.cumsum([0] + _SIZES).tolist()
_TOTAL = int(_OFFS[-1])
_N = 262144
_F = _NUM_LEVELS * _LEVEL_DIM
_HASHED = [r ** 3 > _T for r in _RES]
_P2 = int(np.int32(np.uint32(2654435761)))
_P3 = int(np.int32(np.uint32(805459861)))
_HMASK = _T - 1

_NC = 2
_NS = 16
_NW = _NC * _NS
_PPW = _N // _NW
_BP = 512
_NB = _PPW // _BP
_NG = _BP // 16
_GW = 8 * 16 * _LEVEL_DIM

_PIB = lax.GatherScatterMode.PROMISE_IN_BOUNDS


def _tec_body(pts_h, emb_h, out_h,
              xb, yb, zb, idxb, wb, rowsb, outb,
              scale_sm, resm1_sm, res_sm, res2_sm, off_sm, hash_sm, sem):
    wid = lax.axis_index("s") * _NC + lax.axis_index("c")

    iota = lax.iota(jnp.int32, 16)
    rep4 = [(iota >> 2) + 4 * q for q in range(4)]
    mod4 = iota & 3

    for l in range(_NUM_LEVELS):
        scale_sm[l] = jnp.float32(_RES[l] - 1)
        resm1_sm[l] = jnp.int32(_RES[l] - 1)
        res_sm[l] = jnp.int32(_RES[l])
        res2_sm[l] = jnp.int32(_RES[l] * _RES[l])
        off_sm[l] = jnp.int32(_OFFS[l])
        hash_sm[l] = jnp.int32(1 if _HASHED[l] else 0)

    def block_body(blk, _):
        base = wid * _PPW + blk * _BP
        pltpu.sync_copy(pts_h.at[pl.ds(base, _BP)], xb)
        pltpu.sync_copy(pts_h.at[pl.ds(_N + base, _BP)], yb)
        pltpu.sync_copy(pts_h.at[pl.ds(2 * _N + base, _BP)], zb)

        def clip_body(g, _):
            o = g * 16
            for b in (xb, yb, zb):
                b[pl.ds(o, 16)] = jnp.minimum(
                    jnp.maximum(b[pl.ds(o, 16)], 0.0), 1.0)
            return _

        lax.fori_loop(0, _NG, clip_body, None)

        def level_body(l, _):
            scale = scale_sm[l]
            resm1 = resm1_sm[l]
            res = res_sm[l]
            res2 = res2_sm[l]
            lvl_off = off_sm[l]
            hashed = hash_sm[l] != 0

            def passa(g, _):
                o = g * 16
                px = xb[pl.ds(o, 16)] * scale
                py = yb[pl.ds(o, 16)] * scale
                pz = zb[pl.ds(o, 16)] * scale
                xi = px.astype(jnp.int32)
                yi = py.astype(jnp.int32)
                zi = pz.astype(jnp.int32)
                fx = px - xi.astype(jnp.float32)
                fy = py - yi.astype(jnp.float32)
                fz = pz - zi.astype(jnp.float32)
                x1 = jnp.minimum(xi + 1, resm1)
                y1 = jnp.minimum(yi + 1, resm1)
                z1 = jnp.minimum(zi + 1, resm1)
                gx = (1.0 - fx, fx)
                gy = (1.0 - fy, fy)
                gz = (1.0 - fz, fz)
                cxs = (xi, x1)
                cys = (yi, y1)
                czs = (zi, z1)
                c = 0
                for bi in (0, 1):
                    for bj in (0, 1):
                        for bk in (0, 1):
                            cx, cy, cz = cxs[bi], cys[bj], czs[bk]
                            idx_d = cx + cy * res + cz * res2
                            idx_h = (cx ^ (cy * _P2) ^ (cz * _P3)) & _HMASK
                            idx = jnp.where(hashed, idx_h, idx_d) + lvl_off
                            w = gx[bi] * gy[bj] * gz[bk]
                            widx = idx * _LEVEL_DIM
                            for q in range(4):
                                wq = jnp.take_along_axis(
                                    widx, rep4[q], axis=0, mode=_PIB)
                                p = c * 64 + q * 16
                                idxb[g * 4 + p // 128,
                                     pl.ds(p % 128, 16)] = wq + mod4
                            wb[pl.ds(g * 128 + c * 16, 16)] = w
                            c += 1
                return _

            lax.fori_loop(0, _NG, passa, None)

            descs = [
                pltpu.async_copy(emb_h.at[idxb.at[j]],
                                 rowsb.at[pl.ds(j * 128, 128)], sem)
                for j in range(_NG * 4)
            ]
            for d in descs:
                d.wait()

            def passb(g, _):
                acc = [None] * 4
                for c in range(8):
                    wv = wb[pl.ds(g * 128 + c * 16, 16)]
                    for q in range(4):
                        v = rowsb[pl.ds(g * _GW + c * 64 + q * 16, 16)]
                        wrep = jnp.take_along_axis(
                            wv, rep4[q], axis=0, mode=_PIB)
                        t = v * wrep
                        acc[q] = t if acc[q] is None else acc[q] + t
                obase = l * (_BP * _LEVEL_DIM) + g * 64
                for q in range(4):
                    outb[pl.ds(obase + q * 16, 16)] = acc[q]
                return _

            lax.fori_loop(0, _NG, passb, None)
            return _

        lax.fori_loop(0, _NUM_LEVELS, level_body, None)
        for l in range(_NUM_LEVELS):
            pltpu.sync_copy(
                outb.at[pl.ds(l * _BP * _LEVEL_DIM, _BP * _LEVEL_DIM)],
                out_h.at[pl.ds(l * _N * _LEVEL_DIM + base * _LEVEL_DIM,
                               _BP * _LEVEL_DIM)])
        return _

    lax.fori_loop(0, _NB, block_body, None)


@jax.jit
def kernel(points_3D, embeddings):
    pts_t = jnp.reshape(jnp.transpose(points_3D), (-1,))
    emb_flat = jnp.reshape(embeddings, (-1,))
    mesh = plsc.VectorSubcoreMesh(core_axis_name="c", subcore_axis_name="s")
    run = pl.kernel(
        _tec_body,
        out_type=jax.ShapeDtypeStruct((_NUM_LEVELS * _N * _LEVEL_DIM,),
                                      jnp.float32),
        mesh=mesh,
        scratch_types=[
            pltpu.VMEM((_BP,), jnp.float32),
            pltpu.VMEM((_BP,), jnp.float32),
            pltpu.VMEM((_BP,), jnp.float32),
            pltpu.VMEM((_NG * 4, 128), jnp.int32),
            pltpu.VMEM((_NG * 128,), jnp.float32),
            pltpu.VMEM((_NG * _GW,), jnp.float32),
            pltpu.VMEM((_NUM_LEVELS * _BP * _LEVEL_DIM,), jnp.float32),
            pltpu.SMEM((_NUM_LEVELS,), jnp.float32),
            pltpu.SMEM((_NUM_LEVELS,), jnp.int32),
            pltpu.SMEM((_NUM_LEVELS,), jnp.int32),
            pltpu.SMEM((_NUM_LEVELS,), jnp.int32),
            pltpu.SMEM((_NUM_LEVELS,), jnp.int32),
            pltpu.SMEM((_NUM_LEVELS,), jnp.int32),
            pltpu.SemaphoreType.DMA,
        ],
    )
    out_lvl = run(pts_t, emb_flat)
    return jnp.reshape(
        jnp.transpose(
            jnp.reshape(out_lvl, (_NUM_LEVELS, _N, _LEVEL_DIM)), (1, 0, 2)),
        (_N, _F))


# speedup vs baseline: 1.7165x; 1.1817x over previous
import jax
import jax.numpy as jnp
import numpy as np
from jax import lax
from jax.experimental import pallas as pl
from jax.experimental.pallas import tpu as pltpu
from jax.experimental.pallas import tpu_sc as plsc

_NUM_LEVELS = 16
_LEVEL_DIM = 4
_LOG2_T = 19
_T = 2 ** _LOG2_T
_BASE_RES = 16
_MAX_RES = 2048
_GROWTH = np.exp((np.log(_MAX_RES) - np.log(_BASE_RES)) / (_NUM_LEVELS - 1))
_RES = [int(np.floor(_BASE_RES * _GROWTH ** l)) + 1 for l in range(_NUM_LEVELS)]
_SIZES = [min(_T, r ** 3) for r in _RES]
_OFFS = np.cumsum([0] + _SIZES).tolist()
_TOTAL = int(_OFFS[-1])
_N = 262144
_F = _NUM_LEVELS * _LEVEL_DIM
_HASHED = [r ** 3 > _T for r in _RES]
_P2 = int(np.int32(np.uint32(2654435761)))
_P3 = int(np.int32(np.uint32(805459861)))
_HMASK = _T - 1

_NC = 2
_NS = 16
_NW = _NC * _NS
_PPW = _N // _NW
_BP = 512
_NB = _PPW // _BP
_NG = _BP // 16
_GW = 8 * 16 * _LEVEL_DIM

_PIB = lax.GatherScatterMode.PROMISE_IN_BOUNDS


def _tec_body(pts_h, emb_h, out_h,
              pb, xb, yb, zb, idxb, wb, rowsb, outb, outb2,
              scale_sm, resm1_sm, res_sm, res2_sm, off_sm, hash_sm, sem):
    wid = lax.axis_index("s") * _NC + lax.axis_index("c")

    iota = lax.iota(jnp.int32, 16)
    rep4 = [(iota >> 2) + 4 * q for q in range(4)]
    mod4 = iota & 3
    hi2 = iota >> 2
    j4 = [mod4 + 4 * j for j in range(4)]

    for l in range(_NUM_LEVELS):
        scale_sm[l] = jnp.float32(_RES[l] - 1)
        resm1_sm[l] = jnp.int32(_RES[l] - 1)
        res_sm[l] = jnp.int32(_RES[l])
        res2_sm[l] = jnp.int32(_RES[l] * _RES[l])
        off_sm[l] = jnp.int32(_OFFS[l])
        hash_sm[l] = jnp.int32(1 if _HASHED[l] else 0)

    def block_body(blk, _):
        base = wid * _PPW + blk * _BP
        pltpu.sync_copy(pts_h.at[pl.ds(base * 3, _BP * 3)], pb)

        def deint_body(g, _):
            o = g * 16
            vs = [pb[pl.ds(g * 48 + s * 16, 16)] for s in range(3)]
            for d, b in enumerate((xb, yb, zb)):
                fp = iota * 3 + d
                sv = fp >> 4
                lv = fp & 15
                ts = [jnp.take_along_axis(v, lv, axis=0, mode=_PIB)
                      for v in vs]
                v = jnp.where(sv == 2, ts[2],
                              jnp.where(sv == 1, ts[1], ts[0]))
                b[pl.ds(o, 16)] = jnp.minimum(jnp.maximum(v, 0.0), 1.0)
            return _

        lax.fori_loop(0, _NG, deint_body, None)

        def level_body(l, _):
            scale = scale_sm[l]
            resm1 = resm1_sm[l]
            res = res_sm[l]
            res2 = res2_sm[l]
            lvl_off = off_sm[l]
            hashed = hash_sm[l] != 0

            def passa(g, _):
                o = g * 16
                px = xb[pl.ds(o, 16)] * scale
                py = yb[pl.ds(o, 16)] * scale
                pz = zb[pl.ds(o, 16)] * scale
                xi = px.astype(jnp.int32)
                yi = py.astype(jnp.int32)
                zi = pz.astype(jnp.int32)
                fx = px - xi.astype(jnp.float32)
                fy = py - yi.astype(jnp.float32)
                fz = pz - zi.astype(jnp.float32)
                x1 = jnp.minimum(xi + 1, resm1)
                y1 = jnp.minimum(yi + 1, resm1)
                z1 = jnp.minimum(zi + 1, resm1)
                gx = (1.0 - fx, fx)
                gy = (1.0 - fy, fy)
                gz = (1.0 - fz, fz)
                cxs = (xi, x1)
                cys = (yi, y1)
                czs = (zi, z1)
                c = 0
                for bi in (0, 1):
                    for bj in (0, 1):
                        for bk in (0, 1):
                            cx, cy, cz = cxs[bi], cys[bj], czs[bk]
                            idx_d = cx + cy * res + cz * res2
                            idx_h = (cx ^ (cy * _P2) ^ (cz * _P3)) & _HMASK
                            idx = jnp.where(hashed, idx_h, idx_d) + lvl_off
                            w = gx[bi] * gy[bj] * gz[bk]
                            widx = idx * _LEVEL_DIM
                            for q in range(4):
                                wq = jnp.take_along_axis(
                                    widx, rep4[q], axis=0, mode=_PIB)
                                p = c * 64 + q * 16
                                idxb[g * 4 + p // 128,
                                     pl.ds(p % 128, 16)] = wq + mod4
                            wb[pl.ds(g * 128 + c * 16, 16)] = w
                            c += 1
                return _

            lax.fori_loop(0, _NG, passa, None)

            descs = [
                pltpu.async_copy(emb_h.at[idxb.at[j]],
                                 rowsb.at[pl.ds(j * 128, 128)], sem)
                for j in range(_NG * 4)
            ]
            for d in descs:
                d.wait()

            def passb(g, _):
                acc = [None] * 4
                for c in range(8):
                    wv = wb[pl.ds(g * 128 + c * 16, 16)]
                    for q in range(4):
                        v = rowsb[pl.ds(g * _GW + c * 64 + q * 16, 16)]
                        wrep = jnp.take_along_axis(
                            wv, rep4[q], axis=0, mode=_PIB)
                        t = v * wrep
                        acc[q] = t if acc[q] is None else acc[q] + t
                obase = l * (_BP * _LEVEL_DIM) + g * 64
                for q in range(4):
                    outb[pl.ds(obase + q * 16, 16)] = acc[q]
                return _

            lax.fori_loop(0, _NG, passb, None)
            return _

        lax.fori_loop(0, _NUM_LEVELS, level_body, None)

        def merge_body(b, _):
            vs = [outb[pl.ds(l * _BP * _LEVEL_DIM + b * 16, 16)]
                  for l in range(_NUM_LEVELS)]
            for j in range(4):
                for m in range(4):
                    ts = [jnp.take_along_axis(vs[4 * m + s], j4[j], axis=0,
                                              mode=_PIB) for s in range(4)]
                    v = jnp.where(hi2 == 1, ts[1], ts[0])
                    v = jnp.where(hi2 == 2, ts[2], v)
                    v = jnp.where(hi2 == 3, ts[3], v)
                    outb2[pl.ds((4 * b + j) * _F + 16 * m, 16)] = v
            return _

        lax.fori_loop(0, _BP // 4, merge_body, None)
        pltpu.sync_copy(outb2, out_h.at[pl.ds(base * _F, _BP * _F)])
        return _

    lax.fori_loop(0, _NB, block_body, None)


@jax.jit
def kernel(points_3D, embeddings):
    pts_t = jnp.reshape(points_3D, (-1,))
    emb_flat = jnp.reshape(embeddings, (-1,))
    mesh = plsc.VectorSubcoreMesh(core_axis_name="c", subcore_axis_name="s")
    run = pl.kernel(
        _tec_body,
        out_type=jax.ShapeDtypeStruct((_N * _F,), jnp.float32),
        mesh=mesh,
        scratch_types=[
            pltpu.VMEM((_BP * 3,), jnp.float32),
            pltpu.VMEM((_BP,), jnp.float32),
            pltpu.VMEM((_BP,), jnp.float32),
            pltpu.VMEM((_BP,), jnp.float32),
            pltpu.VMEM((_NG * 4, 128), jnp.int32),
            pltpu.VMEM((_NG * 128,), jnp.float32),
            pltpu.VMEM((_NG * _GW,), jnp.float32),
            pltpu.VMEM((_BP * _F,), jnp.float32),
            pltpu.VMEM((_BP * _F,), jnp.float32),
            pltpu.SMEM((_NUM_LEVELS,), jnp.float32),
            pltpu.SMEM((_NUM_LEVELS,), jnp.int32),
            pltpu.SMEM((_NUM_LEVELS,), jnp.int32),
            pltpu.SMEM((_NUM_LEVELS,), jnp.int32),
            pltpu.SMEM((_NUM_LEVELS,), jnp.int32),
            pltpu.SMEM((_NUM_LEVELS,), jnp.int32),
            pltpu.SemaphoreType.DMA,
        ],
    )
    out_flat = run(pts_t, emb_flat)
    return jnp.reshape(out_flat, (_N, _F))


# speedup vs baseline: 1.7367x; 1.0118x over previous
import jax
import jax.numpy as jnp
import numpy as np
from jax import lax
from jax.experimental import pallas as pl
from jax.experimental.pallas import tpu as pltpu
from jax.experimental.pallas import tpu_sc as plsc

_NUM_LEVELS = 16
_LEVEL_DIM = 4
_LOG2_T = 19
_T = 2 ** _LOG2_T
_BASE_RES = 16
_MAX_RES = 2048
_GROWTH = np.exp((np.log(_MAX_RES) - np.log(_BASE_RES)) / (_NUM_LEVELS - 1))
_RES = [int(np.floor(_BASE_RES * _GROWTH ** l)) + 1 for l in range(_NUM_LEVELS)]
_SIZES = [min(_T, r ** 3) for r in _RES]
_OFFS = np.cumsum([0] + _SIZES).tolist()
_TOTAL = int(_OFFS[-1])
_N = 262144
_F = _NUM_LEVELS * _LEVEL_DIM
_HASHED = [r ** 3 > _T for r in _RES]
_P2 = int(np.int32(np.uint32(2654435761)))
_P3 = int(np.int32(np.uint32(805459861)))
_HMASK = _T - 1

_NC = 2
_NS = 16
_NW = _NC * _NS
_PPW = _N // _NW
_BP = 512
_NB = _PPW // _BP
_NG = _BP // 16
_GW = 8 * 16 * _LEVEL_DIM

_PIB = lax.GatherScatterMode.PROMISE_IN_BOUNDS


def _tec_body(pts_h, emb_h, out_h,
              pb, xb, yb, zb, idxb, wb, rowsb, outb, outb2,
              scale_sm, resm1_sm, res_sm, res2_sm, off_sm, hash_sm, sem):
    wid = lax.axis_index("s") * _NC + lax.axis_index("c")

    iota = lax.iota(jnp.int32, 16)
    rep4 = [(iota >> 2) + 4 * q for q in range(4)]
    mod4 = iota & 3
    hi2 = iota >> 2
    j4 = [mod4 + 4 * j for j in range(4)]

    for l in range(_NUM_LEVELS):
        scale_sm[l] = jnp.float32(_RES[l] - 1)
        resm1_sm[l] = jnp.int32(_RES[l] - 1)
        res_sm[l] = jnp.int32(_RES[l])
        res2_sm[l] = jnp.int32(_RES[l] * _RES[l])
        off_sm[l] = jnp.int32(_OFFS[l])
        hash_sm[l] = jnp.int32(1 if _HASHED[l] else 0)

    def block_body(blk, _):
        base = wid * _PPW + blk * _BP
        pltpu.sync_copy(pts_h.at[pl.ds(base * 3, _BP * 3)], pb)

        def deint_body(g, _):
            o = g * 16
            vs = [pb[pl.ds(g * 48 + s * 16, 16)] for s in range(3)]
            for d, b in enumerate((xb, yb, zb)):
                fp = iota * 3 + d
                sv = fp >> 4
                lv = fp & 15
                ts = [jnp.take_along_axis(v, lv, axis=0, mode=_PIB)
                      for v in vs]
                v = jnp.where(sv == 2, ts[2],
                              jnp.where(sv == 1, ts[1], ts[0]))
                b[pl.ds(o, 16)] = jnp.minimum(jnp.maximum(v, 0.0), 1.0)
            return _

        lax.fori_loop(0, _NG, deint_body, None)

        def level_body(l, _):
            scale = scale_sm[l]
            resm1 = resm1_sm[l]
            res = res_sm[l]
            res2 = res2_sm[l]
            lvl_off = off_sm[l]
            hashed = hash_sm[l] != 0

            def passa(g, _):
                o = g * 16
                px = xb[pl.ds(o, 16)] * scale
                py = yb[pl.ds(o, 16)] * scale
                pz = zb[pl.ds(o, 16)] * scale
                xi = px.astype(jnp.int32)
                yi = py.astype(jnp.int32)
                zi = pz.astype(jnp.int32)
                fx = px - xi.astype(jnp.float32)
                fy = py - yi.astype(jnp.float32)
                fz = pz - zi.astype(jnp.float32)
                x1 = jnp.minimum(xi + 1, resm1)
                y1 = jnp.minimum(yi + 1, resm1)
                z1 = jnp.minimum(zi + 1, resm1)
                gx = (1.0 - fx, fx)
                gy = (1.0 - fy, fy)
                gz = (1.0 - fz, fz)
                cxs = (xi, x1)
                cys = (yi, y1)
                czs = (zi, z1)
                c = 0
                for bi in (0, 1):
                    for bj in (0, 1):
                        for bk in (0, 1):
                            cx, cy, cz = cxs[bi], cys[bj], czs[bk]
                            idx_d = cx + cy * res + cz * res2
                            idx_h = (cx ^ (cy * _P2) ^ (cz * _P3)) & _HMASK
                            idx = jnp.where(hashed, idx_h, idx_d) + lvl_off
                            w = gx[bi] * gy[bj] * gz[bk]
                            widx = idx * _LEVEL_DIM
                            for q in range(4):
                                wq = jnp.take_along_axis(
                                    widx, rep4[q], axis=0, mode=_PIB)
                                p = c * 64 + q * 16
                                idxb[g * 4 + p // 128,
                                     pl.ds(p % 128, 16)] = wq + mod4
                            wb[pl.ds(g * 128 + c * 16, 16)] = w
                            c += 1
                return _

            lax.fori_loop(0, _NG, passa, None)

            descs = [
                pltpu.async_copy(emb_h.at[idxb.at[j]],
                                 rowsb.at[pl.ds(j * 128, 128)], sem)
                for j in range(_NG * 4)
            ]
            for d in descs:
                d.wait()

            def passb(g, _):
                acc = [None] * 4
                for c in range(8):
                    wv = wb[pl.ds(g * 128 + c * 16, 16)]
                    for q in range(4):
                        v = rowsb[pl.ds(g * _GW + c * 64 + q * 16, 16)]
                        wrep = jnp.take_along_axis(
                            wv, rep4[q], axis=0, mode=_PIB)
                        t = v * wrep
                        acc[q] = t if acc[q] is None else acc[q] + t
                obase = l * (_BP * _LEVEL_DIM) + g * 64
                for q in range(4):
                    outb[pl.ds(obase + q * 16, 16)] = acc[q]
                return _

            lax.fori_loop(0, _NG, passb, None)
            return _

        lax.fori_loop(0, _NUM_LEVELS, level_body, None)

        for l in range(_NUM_LEVELS):
            def merge_body(g, _, l=l):
                qs = [outb[pl.ds(l * _BP * _LEVEL_DIM + g * 64 + q * 16, 16)]
                      for q in range(4)]
                for ff in range(_LEVEL_DIM):
                    ts = [jnp.take_along_axis(q, mod4 * 4 + ff, axis=0,
                                              mode=_PIB) for q in qs]
                    v = jnp.where(hi2 == 1, ts[1], ts[0])
                    v = jnp.where(hi2 == 2, ts[2], v)
                    v = jnp.where(hi2 == 3, ts[3], v)
                    outb2[l * _LEVEL_DIM + ff, pl.ds(g * 16, 16)] = v
                return _

            lax.fori_loop(0, _NG, merge_body, None)
        for f0 in range(0, _F, 8):
            pltpu.sync_copy(outb2.at[pl.ds(f0, 8), :],
                            out_h.at[pl.ds(f0, 8), pl.ds(base, _BP)])
        return _

    lax.fori_loop(0, _NB, block_body, None)


@jax.jit
def kernel(points_3D, embeddings):
    pts_t = jnp.reshape(points_3D, (-1,))
    emb_flat = jnp.reshape(embeddings, (-1,))
    mesh = plsc.VectorSubcoreMesh(core_axis_name="c", subcore_axis_name="s")
    run = pl.kernel(
        _tec_body,
        out_type=jax.ShapeDtypeStruct((_F, _N), jnp.float32),
        mesh=mesh,
        scratch_types=[
            pltpu.VMEM((_BP * 3,), jnp.float32),
            pltpu.VMEM((_BP,), jnp.float32),
            pltpu.VMEM((_BP,), jnp.float32),
            pltpu.VMEM((_BP,), jnp.float32),
            pltpu.VMEM((_NG * 4, 128), jnp.int32),
            pltpu.VMEM((_NG * 128,), jnp.float32),
            pltpu.VMEM((_NG * _GW,), jnp.float32),
            pltpu.VMEM((_BP * _F,), jnp.float32),
            pltpu.VMEM((_F, _BP), jnp.float32),
            pltpu.SMEM((_NUM_LEVELS,), jnp.float32),
            pltpu.SMEM((_NUM_LEVELS,), jnp.int32),
            pltpu.SMEM((_NUM_LEVELS,), jnp.int32),
            pltpu.SMEM((_NUM_LEVELS,), jnp.int32),
            pltpu.SMEM((_NUM_LEVELS,), jnp.int32),
            pltpu.SMEM((_NUM_LEVELS,), jnp.int32),
            pltpu.SemaphoreType.DMA,
        ],
    )
    return jnp.transpose(run(pts_t, emb_flat))


# speedup vs baseline: 3.9263x; 2.2608x over previous
import jax
import jax.numpy as jnp
import numpy as np
from jax import lax
from jax.experimental import pallas as pl
from jax.experimental.pallas import tpu as pltpu
from jax.experimental.pallas import tpu_sc as plsc

_NUM_LEVELS = 16
_LEVEL_DIM = 4
_LOG2_T = 19
_T = 2 ** _LOG2_T
_BASE_RES = 16
_MAX_RES = 2048
_GROWTH = np.exp((np.log(_MAX_RES) - np.log(_BASE_RES)) / (_NUM_LEVELS - 1))
_RES = [int(np.floor(_BASE_RES * _GROWTH ** l)) + 1 for l in range(_NUM_LEVELS)]
_SIZES = [min(_T, r ** 3) for r in _RES]
_OFFS = np.cumsum([0] + _SIZES).tolist()
_TOTAL = int(_OFFS[-1])
_N = 262144
_F = _NUM_LEVELS * _LEVEL_DIM
_HASHED = [r ** 3 > _T for r in _RES]
_P2 = int(np.int32(np.uint32(2654435761)))
_P3 = int(np.int32(np.uint32(805459861)))
_HMASK = _T - 1

_NC = 2
_NS = 16
_NW = _NC * _NS
_PPW = _N // _NW
_BP = 512
_NB = _PPW // _BP
_NG = _BP // 16
_GW = 8 * 16 * _LEVEL_DIM

_PIB = lax.GatherScatterMode.PROMISE_IN_BOUNDS


def _tec_body(pts_h, emb_h, out_h,
              pb, xb, yb, zb, idxb, wb, rowsb, outb, outb2,
              scale_sm, resm1_sm, res_sm, res2_sm, off_sm, hash_sm, sem):
    wid = lax.axis_index("s") * _NC + lax.axis_index("c")

    iota = lax.iota(jnp.int32, 16)
    rep4 = [(iota >> 2) + 4 * q for q in range(4)]
    mod4 = iota & 3
    mod4_128 = mod4 * 128
    hi2 = iota >> 2
    j4 = [mod4 + 4 * j for j in range(4)]

    for l in range(_NUM_LEVELS):
        scale_sm[l] = jnp.float32(_RES[l] - 1)
        resm1_sm[l] = jnp.int32(_RES[l] - 1)
        res_sm[l] = jnp.int32(_RES[l])
        res2_sm[l] = jnp.int32(_RES[l] * _RES[l])
        off_sm[l] = jnp.int32(_OFFS[l])
        hash_sm[l] = jnp.int32(1 if _HASHED[l] else 0)

    def block_body(blk, _):
        base = wid * _PPW + blk * _BP
        pltpu.sync_copy(pts_h.at[pl.ds(base * 3, _BP * 3)], pb)

        def deint_body(g, _):
            o = g * 16
            vs = [pb[pl.ds(g * 48 + s * 16, 16)] for s in range(3)]
            for d, b in enumerate((xb, yb, zb)):
                fp = iota * 3 + d
                sv = fp >> 4
                lv = fp & 15
                ts = [jnp.take_along_axis(v, lv, axis=0, mode=_PIB)
                      for v in vs]
                v = jnp.where(sv == 2, ts[2],
                              jnp.where(sv == 1, ts[1], ts[0]))
                b[pl.ds(o, 16)] = jnp.minimum(jnp.maximum(v, 0.0), 1.0)
            return _

        lax.fori_loop(0, _NG, deint_body, None)

        def level_body(l, _):
            scale = scale_sm[l]
            resm1 = resm1_sm[l]
            res = res_sm[l]
            res2 = res2_sm[l]
            lvl_off = off_sm[l]
            hashed = hash_sm[l] != 0

            def passa(g, _):
                o = g * 16
                px = xb[pl.ds(o, 16)] * scale
                py = yb[pl.ds(o, 16)] * scale
                pz = zb[pl.ds(o, 16)] * scale
                xi = px.astype(jnp.int32)
                yi = py.astype(jnp.int32)
                zi = pz.astype(jnp.int32)
                fx = px - xi.astype(jnp.float32)
                fy = py - yi.astype(jnp.float32)
                fz = pz - zi.astype(jnp.float32)
                x1 = jnp.minimum(xi + 1, resm1)
                y1 = jnp.minimum(yi + 1, resm1)
                z1 = jnp.minimum(zi + 1, resm1)
                gx = (1.0 - fx, fx)
                gy = (1.0 - fy, fy)
                gz = (1.0 - fz, fz)
                cxs = (xi, x1)
                cys = (yi, y1)
                czs = (zi, z1)
                c = 0
                for bi in (0, 1):
                    for bj in (0, 1):
                        for bk in (0, 1):
                            cx, cy, cz = cxs[bi], cys[bj], czs[bk]
                            idx_d = cx + cy * res + cz * res2
                            idx_h = (cx ^ (cy * _P2) ^ (cz * _P3)) & _HMASK
                            idx = jnp.where(hashed, idx_h, idx_d) + lvl_off
                            w = gx[bi] * gy[bj] * gz[bk]
                            widx = ((idx & -128) << 2) | (idx & 127)
                            for q in range(4):
                                wq = jnp.take_along_axis(
                                    widx, rep4[q], axis=0, mode=_PIB)
                                p = c * 64 + q * 16
                                idxb[g * 4 + p // 128,
                                     pl.ds(p % 128, 16)] = wq + mod4_128
                            wb[pl.ds(g * 128 + c * 16, 16)] = w
                            c += 1
                return _

            lax.fori_loop(0, _NG, passa, None)

            descs = [
                pltpu.async_copy(emb_h.at[idxb.at[j]],
                                 rowsb.at[pl.ds(j * 128, 128)], sem)
                for j in range(_NG * 4)
            ]
            for d in descs:
                d.wait()

            def passb(g, _):
                acc = [None] * 4
                for c in range(8):
                    wv = wb[pl.ds(g * 128 + c * 16, 16)]
                    for q in range(4):
                        v = rowsb[pl.ds(g * _GW + c * 64 + q * 16, 16)]
                        wrep = jnp.take_along_axis(
                            wv, rep4[q], axis=0, mode=_PIB)
                        t = v * wrep
                        acc[q] = t if acc[q] is None else acc[q] + t
                obase = l * (_BP * _LEVEL_DIM) + g * 64
                for q in range(4):
                    outb[pl.ds(obase + q * 16, 16)] = acc[q]
                return _

            lax.fori_loop(0, _NG, passb, None)
            return _

        lax.fori_loop(0, _NUM_LEVELS, level_body, None)

        for l in range(_NUM_LEVELS):
            def merge_body(g, _, l=l):
                qs = [outb[pl.ds(l * _BP * _LEVEL_DIM + g * 64 + q * 16, 16)]
                      for q in range(4)]
                for ff in range(_LEVEL_DIM):
                    ts = [jnp.take_along_axis(q, mod4 * 4 + ff, axis=0,
                                              mode=_PIB) for q in qs]
                    v = jnp.where(hi2 == 1, ts[1], ts[0])
                    v = jnp.where(hi2 == 2, ts[2], v)
                    v = jnp.where(hi2 == 3, ts[3], v)
                    outb2[l * _LEVEL_DIM + ff, pl.ds(g * 16, 16)] = v
                return _

            lax.fori_loop(0, _NG, merge_body, None)
        for f0 in range(0, _F, 8):
            pltpu.sync_copy(outb2.at[pl.ds(f0, 8), :],
                            out_h.at[pl.ds(f0, 8), pl.ds(base, _BP)])
        return _

    lax.fori_loop(0, _NB, block_body, None)


@jax.jit
def kernel(points_3D, embeddings):
    pts_t = jnp.reshape(points_3D, (-1,))
    nblk = (_TOTAL + 127) // 128
    padded = jnp.pad(embeddings, ((0, nblk * 128 - _TOTAL), (0, 0)))
    emb_flat = jnp.reshape(
        jnp.transpose(jnp.reshape(padded, (nblk, 128, _LEVEL_DIM)),
                      (0, 2, 1)), (-1,))
    mesh = plsc.VectorSubcoreMesh(core_axis_name="c", subcore_axis_name="s")
    run = pl.kernel(
        _tec_body,
        out_type=jax.ShapeDtypeStruct((_F, _N), jnp.float32),
        mesh=mesh,
        scratch_types=[
            pltpu.VMEM((_BP * 3,), jnp.float32),
            pltpu.VMEM((_BP,), jnp.float32),
            pltpu.VMEM((_BP,), jnp.float32),
            pltpu.VMEM((_BP,), jnp.float32),
            pltpu.VMEM((_NG * 4, 128), jnp.int32),
            pltpu.VMEM((_NG * 128,), jnp.float32),
            pltpu.VMEM((_NG * _GW,), jnp.float32),
            pltpu.VMEM((_BP * _F,), jnp.float32),
            pltpu.VMEM((_F, _BP), jnp.float32),
            pltpu.SMEM((_NUM_LEVELS,), jnp.float32),
            pltpu.SMEM((_NUM_LEVELS,), jnp.int32),
            pltpu.SMEM((_NUM_LEVELS,), jnp.int32),
            pltpu.SMEM((_NUM_LEVELS,), jnp.int32),
            pltpu.SMEM((_NUM_LEVELS,), jnp.int32),
            pltpu.SMEM((_NUM_LEVELS,), jnp.int32),
            pltpu.SemaphoreType.DMA,
        ],
    )
    return jnp.transpose(run(pts_t, emb_flat))


# speedup vs baseline: 3.9320x; 1.0015x over previous
import jax
import jax.numpy as jnp
import numpy as np
from jax import lax
from jax.experimental import pallas as pl
from jax.experimental.pallas import tpu as pltpu
from jax.experimental.pallas import tpu_sc as plsc

_NUM_LEVELS = 16
_LEVEL_DIM = 4
_LOG2_T = 19
_T = 2 ** _LOG2_T
_BASE_RES = 16
_MAX_RES = 2048
_GROWTH = np.exp((np.log(_MAX_RES) - np.log(_BASE_RES)) / (_NUM_LEVELS - 1))
_RES = [int(np.floor(_BASE_RES * _GROWTH ** l)) + 1 for l in range(_NUM_LEVELS)]
_SIZES = [min(_T, r ** 3) for r in _RES]
_OFFS = np.cumsum([0] + _SIZES).tolist()
_TOTAL = int(_OFFS[-1])
_N = 262144
_F = _NUM_LEVELS * _LEVEL_DIM
_HASHED = [r ** 3 > _T for r in _RES]
_P2 = int(np.int32(np.uint32(2654435761)))
_P3 = int(np.int32(np.uint32(805459861)))
_HMASK = _T - 1

_NC = 2
_NS = 16
_NW = _NC * _NS
_PPW = _N // _NW
_BP = 512
_NB = _PPW // _BP
_NG = _BP // 16
_GW = 8 * 16 * _LEVEL_DIM

_PIB = lax.GatherScatterMode.PROMISE_IN_BOUNDS


def _tec_body(pts_h, emb_h, out_h,
              pb, xb, yb, zb, idxb, wb, rowsb, outb, outb2,
              scale_sm, resm1_sm, res_sm, res2_sm, off_sm, hash_sm, sem):
    wid = lax.axis_index("s") * _NC + lax.axis_index("c")

    iota = lax.iota(jnp.int32, 16)
    rep4 = [(iota >> 2) + 4 * q for q in range(4)]
    mod4 = iota & 3
    mod4_128 = mod4 * 128
    hi2 = iota >> 2
    j4 = [mod4 + 4 * j for j in range(4)]

    for l in range(_NUM_LEVELS):
        scale_sm[l] = jnp.float32(_RES[l] - 1)
        resm1_sm[l] = jnp.int32(_RES[l] - 1)
        res_sm[l] = jnp.int32(_RES[l])
        res2_sm[l] = jnp.int32(_RES[l] * _RES[l])
        off_sm[l] = jnp.int32(_OFFS[l])
        hash_sm[l] = jnp.int32(1 if _HASHED[l] else 0)

    def block_body(blk, _):
        base = wid * _PPW + blk * _BP
        pltpu.sync_copy(pts_h.at[pl.ds(base * 3, _BP * 3)], pb)

        def deint_body(g, _):
            o = g * 16
            vs = [pb[pl.ds(g * 48 + s * 16, 16)] for s in range(3)]
            for d, b in enumerate((xb, yb, zb)):
                fp = iota * 3 + d
                sv = fp >> 4
                lv = fp & 15
                ts = [jnp.take_along_axis(v, lv, axis=0, mode=_PIB)
                      for v in vs]
                v = jnp.where(sv == 2, ts[2],
                              jnp.where(sv == 1, ts[1], ts[0]))
                b[pl.ds(o, 16)] = jnp.minimum(jnp.maximum(v, 0.0), 1.0)
            return _

        lax.fori_loop(0, _NG, deint_body, None)

        def level_body(l, _):
            scale = scale_sm[l]
            resm1 = resm1_sm[l]
            res = res_sm[l]
            res2 = res2_sm[l]
            lvl_off = off_sm[l]
            hashed = hash_sm[l] != 0

            def passa(g, _):
                o = g * 16
                px = xb[pl.ds(o, 16)] * scale
                py = yb[pl.ds(o, 16)] * scale
                pz = zb[pl.ds(o, 16)] * scale
                xi = px.astype(jnp.int32)
                yi = py.astype(jnp.int32)
                zi = pz.astype(jnp.int32)
                fx = px - xi.astype(jnp.float32)
                fy = py - yi.astype(jnp.float32)
                fz = pz - zi.astype(jnp.float32)
                x1 = jnp.minimum(xi + 1, resm1)
                y1 = jnp.minimum(yi + 1, resm1)
                z1 = jnp.minimum(zi + 1, resm1)
                gx = (1.0 - fx, fx)
                gy = (1.0 - fy, fy)
                gz = (1.0 - fz, fz)
                cxs = (xi, x1)
                cys = (yi, y1)
                czs = (zi, z1)
                c = 0
                for bi in (0, 1):
                    for bj in (0, 1):
                        for bk in (0, 1):
                            cx, cy, cz = cxs[bi], cys[bj], czs[bk]
                            idx_d = cx + cy * res + cz * res2
                            idx_h = (cx ^ (cy * _P2) ^ (cz * _P3)) & _HMASK
                            idx = jnp.where(hashed, idx_h, idx_d) + lvl_off
                            w = gx[bi] * gy[bj] * gz[bk]
                            widx = ((idx & -128) << 2) | (idx & 127)
                            for q in range(4):
                                wq = jnp.take_along_axis(
                                    widx, rep4[q], axis=0, mode=_PIB)
                                idxb[pl.ds(g * 512 + c * 64 + q * 16,
                                           16)] = wq + mod4_128
                            wb[pl.ds(g * 128 + c * 16, 16)] = w
                            c += 1
                return _

            lax.fori_loop(0, _NG, passa, None)

            descs = [
                pltpu.async_copy(emb_h.at[idxb.at[pl.ds(j * 512, 512)]],
                                 rowsb.at[pl.ds(j * 512, 512)], sem)
                for j in range(_NG)
            ]
            for d in descs:
                d.wait()

            def passb(g, _):
                acc = [None] * 4
                for c in range(8):
                    wv = wb[pl.ds(g * 128 + c * 16, 16)]
                    for q in range(4):
                        v = rowsb[pl.ds(g * _GW + c * 64 + q * 16, 16)]
                        wrep = jnp.take_along_axis(
                            wv, rep4[q], axis=0, mode=_PIB)
                        t = v * wrep
                        acc[q] = t if acc[q] is None else acc[q] + t
                obase = l * (_BP * _LEVEL_DIM) + g * 64
                for q in range(4):
                    outb[pl.ds(obase + q * 16, 16)] = acc[q]
                return _

            lax.fori_loop(0, _NG, passb, None)
            return _

        lax.fori_loop(0, _NUM_LEVELS, level_body, None)

        for l in range(_NUM_LEVELS):
            def merge_body(g, _, l=l):
                qs = [outb[pl.ds(l * _BP * _LEVEL_DIM + g * 64 + q * 16, 16)]
                      for q in range(4)]
                for ff in range(_LEVEL_DIM):
                    ts = [jnp.take_along_axis(q, mod4 * 4 + ff, axis=0,
                                              mode=_PIB) for q in qs]
                    v = jnp.where(hi2 == 1, ts[1], ts[0])
                    v = jnp.where(hi2 == 2, ts[2], v)
                    v = jnp.where(hi2 == 3, ts[3], v)
                    outb2[l * _LEVEL_DIM + ff, pl.ds(g * 16, 16)] = v
                return _

            lax.fori_loop(0, _NG, merge_body, None)
        for f0 in range(0, _F, 8):
            pltpu.sync_copy(outb2.at[pl.ds(f0, 8), :],
                            out_h.at[pl.ds(f0, 8), pl.ds(base, _BP)])
        return _

    lax.fori_loop(0, _NB, block_body, None)


@jax.jit
def kernel(points_3D, embeddings):
    pts_t = jnp.reshape(points_3D, (-1,))
    nblk = (_TOTAL + 127) // 128
    padded = jnp.pad(embeddings, ((0, nblk * 128 - _TOTAL), (0, 0)))
    emb_flat = jnp.reshape(
        jnp.transpose(jnp.reshape(padded, (nblk, 128, _LEVEL_DIM)),
                      (0, 2, 1)), (-1,))
    mesh = plsc.VectorSubcoreMesh(core_axis_name="c", subcore_axis_name="s")
    run = pl.kernel(
        _tec_body,
        out_type=jax.ShapeDtypeStruct((_F, _N), jnp.float32),
        mesh=mesh,
        scratch_types=[
            pltpu.VMEM((_BP * 3,), jnp.float32),
            pltpu.VMEM((_BP,), jnp.float32),
            pltpu.VMEM((_BP,), jnp.float32),
            pltpu.VMEM((_BP,), jnp.float32),
            pltpu.VMEM((_NG * 512,), jnp.int32),
            pltpu.VMEM((_NG * 128,), jnp.float32),
            pltpu.VMEM((_NG * _GW,), jnp.float32),
            pltpu.VMEM((_BP * _F,), jnp.float32),
            pltpu.VMEM((_F, _BP), jnp.float32),
            pltpu.SMEM((_NUM_LEVELS,), jnp.float32),
            pltpu.SMEM((_NUM_LEVELS,), jnp.int32),
            pltpu.SMEM((_NUM_LEVELS,), jnp.int32),
            pltpu.SMEM((_NUM_LEVELS,), jnp.int32),
            pltpu.SMEM((_NUM_LEVELS,), jnp.int32),
            pltpu.SMEM((_NUM_LEVELS,), jnp.int32),
            pltpu.SemaphoreType.DMA,
        ],
    )
    return jnp.transpose(run(pts_t, emb_flat))
